# baseline (speedup 1.0000x reference)
import functools

import jax
import jax.numpy as jnp
from jax import lax
from jax.experimental import pallas as pl
from jax.experimental.pallas import tpu as pltpu
from jax.experimental.pallas import tpu_sc as plsc

E_ROWS = 100000
C_ROWS = 110000
N_CYC = 20000
SPLIT = 50000
BN_EPS = 1e-5

BN = 1000
CH = 10240
SG = 8
PAD_DI = jnp.int32(1 << 30)


@functools.lru_cache(maxsize=None)
def _make_scatter_kernel(ng, n_ch):
    max_cpc = (n_ch + 1) // 2
    zch = CH // 16
    n_strips = ng // SG
    mesh = plsc.VectorSubcoreMesh(core_axis_name="c", subcore_axis_name="s")

    RB = 2048

    def body(src_hbm, si_hbm, di_hbm, out_hbm,
             si2d, di2d, csi, cld2, b0, b1, accum, semg, sems):
        core = lax.axis_index("c")
        sub = lax.axis_index("s")

        bufs = (b0, b1)

        def gat(dbase, buf):
            off = pl.multiple_of(dbase & (RB - 1), 128)
            return pltpu.async_copy(src_hbm.at[csi.at[pl.ds(off, 128)]],
                                    buf, semg)

        def sca(dbase, buf):
            return pltpu.async_copy(buf, accum.at[cld2.at[(dbase >> 7) & 15]],
                                    sems, add=True)

        def burst(dbase):
            hg = gat(dbase, b0)
            hs_prev = None
            for k in range(8):
                hg.wait()
                if hs_prev is not None:
                    hs_prev.wait()
                if k < 7:
                    hg = gat(dbase + (k + 1) * 128, bufs[(k + 1) % 2])
                hs_prev = sca(dbase + k * 128, bufs[k % 2])
            hs_prev.wait()

        def pair(dbase):
            h0 = gat(dbase, b0)
            h1 = gat(dbase + 128, b1)
            h0.wait()
            pltpu.sync_copy(b0, accum.at[cld2.at[(dbase >> 7) & 15]], add=True)
            h1.wait()
            pltpu.sync_copy(b1, accum.at[cld2.at[((dbase >> 7) + 1) & 15]],
                            add=True)

        def single(dbase):
            h0 = gat(dbase, b0)
            h0.wait()
            pltpu.sync_copy(b0, accum.at[cld2.at[(dbase >> 7) & 15]], add=True)

        def chunk_step(i, _):
            c = 2 * i + core
            active = c < n_ch
            base = c * CH

            @pl.when(active)
            def _zero():
                def zrow(r, _):
                    for j in range(8):
                        b0[r, pl.ds(j * 16, 16)] = jnp.zeros((16,), jnp.float32)
                    return 0
                lax.fori_loop(0, 128, zrow, 0)

                hz = [pltpu.async_copy(
                          b0, accum.at[pl.ds(sub * zch + k * 128, 128)], sems)
                      for k in range(zch // 128)]
                for h in hz:
                    h.wait()

            plsc.subcore_barrier()

            @pl.when(active)
            def _accumulate():
                iot = lax.iota(jnp.int32, 16)

                def strip(t, carry):
                    cnt, dptr = carry
                    pltpu.sync_copy(si_hbm.at[sub, pl.ds(t * SG, SG)], si2d)
                    pltpu.sync_copy(di_hbm.at[sub, pl.ds(t * SG, SG)], di2d)

                    def cgrp(g, cnt):
                        for j in range(8):
                            vs = si2d[g, pl.ds(j * 16, 16)]
                            vd = di2d[g, pl.ds(j * 16, 16)]
                            m = (vd >= base) & (vd < base + CH)
                            ones = jnp.where(m, 1, 0)
                            pos = cnt + plsc.cumsum(ones) - 1
                            plsc.store_scatter(csi, [pos & (RB - 1)], vs, mask=m)
                            plsc.store_scatter(
                                cld2, [(pos >> 7) & 15, pos & 127],
                                vd - base, mask=m)
                            cnt = cnt + jnp.sum(ones)
                        return cnt
                    cnt = lax.fori_loop(0, SG, cgrp, cnt + jnp.int32(0))

                    trig = cnt - dptr >= 1024

                    @pl.when(trig)
                    def _d():
                        burst(dptr)
                    return (cnt, dptr + jnp.where(trig, 1024, 0))

                cnt, dptr = lax.fori_loop(0, n_strips, strip,
                                          (jnp.int32(0), jnp.int32(0)))

                for k in range(8):
                    p = cnt + k * 16 + iot
                    plsc.store_scatter(csi, [p & (RB - 1)],
                                       jnp.zeros((16,), jnp.int32))
                    plsc.store_scatter(cld2, [(p >> 7) & 15, p & 127],
                                       jnp.full((16,), CH, jnp.int32))

                npt = (cnt - dptr) // 256

                def drt(p, _):
                    pair(dptr + p * 256)
                    return 0
                lax.fori_loop(0, npt, drt, 0)
                dptr2 = dptr + npt * 256

                def drs(p, _):
                    single(dptr2 + p * 128)
                    return 0
                lax.fori_loop(0, (cnt - dptr2 + 127) // 128, drs, 0)

            plsc.subcore_barrier()

            @pl.when(active)
            def _flush():
                pltpu.sync_copy(accum.at[pl.ds(sub * zch, zch)],
                                out_hbm.at[pl.ds(base + sub * zch, zch)])

            plsc.subcore_barrier()
            return 0

        lax.fori_loop(0, max_cpc, chunk_step, 0)

    return pl.kernel(
        body,
        out_type=jax.ShapeDtypeStruct((n_ch * CH, 128), jnp.float32),
        mesh=mesh,
        compiler_params=pltpu.CompilerParams(needs_layout_passes=False),
        scratch_types=[
            pltpu.VMEM((SG, 128), jnp.int32),
            pltpu.VMEM((SG, 128), jnp.int32),
            pltpu.VMEM((2048,), jnp.int32),
            pltpu.VMEM((16, 128), jnp.int32),
            pltpu.VMEM((128, 128), jnp.float32),
            pltpu.VMEM((128, 128), jnp.float32),
            pltpu.VMEM_SHARED((CH + 8, 128), jnp.float32),
            pltpu.SemaphoreType.DMA,
            pltpu.SemaphoreType.DMA,
        ],
    )


def _scatter_sc(src, si, di, n_out):
    m = si.shape[0]
    mult = 16 * SG * 128
    m_pad = -(-m // mult) * mult
    if m_pad != m:
        si = jnp.concatenate([si, jnp.zeros((m_pad - m,), jnp.int32)])
        di = jnp.concatenate([di, jnp.full((m_pad - m,), PAD_DI, jnp.int32)])
    ng = m_pad // 16 // 128
    n_ch = -(-n_out // CH)
    si3 = si.reshape(16, ng, 128)
    di3 = di.reshape(16, ng, 128)
    k = _make_scatter_kernel(ng, n_ch)
    out = k(src, si3, di3)
    return out[:n_out]


def _linear_bn(x, w, b, scale=None, shift=None):
    n, din = x.shape
    dout = w.shape[1]
    nb = n // BN
    apply_act = scale is not None

    def body(*refs):
        if apply_act:
            x_ref, s_ref, h_ref, w_ref, b_ref, y_ref, st_ref = refs
            xx = jnp.maximum(x_ref[...] * s_ref[...] + h_ref[...], 0.0)
        else:
            x_ref, w_ref, b_ref, y_ref, st_ref = refs
            xx = x_ref[...]
        y = jnp.dot(xx, w_ref[...], preferred_element_type=jnp.float32) + b_ref[...]
        y_ref[...] = y
        st = jnp.stack([jnp.sum(y, axis=0), jnp.sum(y * y, axis=0)], axis=0)

        @pl.when(pl.program_id(0) == 0)
        def _():
            st_ref[...] = st

        @pl.when(pl.program_id(0) != 0)
        def _():
            st_ref[...] += st

    in_specs = [pl.BlockSpec((BN, din), lambda i: (i, 0))]
    args = [x]
    if apply_act:
        in_specs += [pl.BlockSpec((1, din), lambda i: (0, 0)),
                     pl.BlockSpec((1, din), lambda i: (0, 0))]
        args += [scale.reshape(1, din), shift.reshape(1, din)]
    in_specs += [pl.BlockSpec((din, dout), lambda i: (0, 0)),
                 pl.BlockSpec((1, dout), lambda i: (0, 0))]
    args += [w, b.reshape(1, dout)]

    y, st = pl.pallas_call(
        body,
        grid=(nb,),
        in_specs=in_specs,
        out_specs=[pl.BlockSpec((BN, dout), lambda i: (i, 0)),
                   pl.BlockSpec((2, dout), lambda i: (0, 0))],
        out_shape=[jax.ShapeDtypeStruct((n, dout), jnp.float32),
                   jax.ShapeDtypeStruct((2, dout), jnp.float32)],
    )(*args)
    return y, st


def _bn_coeffs(st, n, g, be):
    mu = st[0] / n
    var = st[1] / n - mu * mu
    inv = g * lax.rsqrt(var + BN_EPS)
    return inv, be - mu * inv


def _mlp(params, x):
    n = x.shape[0]
    scale = shift = None
    for (w, b, g, be) in params:
        x, st = _linear_bn(x, w, b, scale, shift)
        scale, shift = _bn_coeffs(st, n, g, be)
    return x, scale, shift


def _affine_relu(x, scale, shift):
    n, d = x.shape

    def body(x_ref, s_ref, h_ref, o_ref):
        o_ref[...] = jnp.maximum(x_ref[...] * s_ref[...] + h_ref[...], 0.0)

    return pl.pallas_call(
        body,
        grid=(n // BN,),
        in_specs=[pl.BlockSpec((BN, d), lambda i: (i, 0)),
                  pl.BlockSpec((1, d), lambda i: (0, 0)),
                  pl.BlockSpec((1, d), lambda i: (0, 0))],
        out_specs=pl.BlockSpec((BN, d), lambda i: (i, 0)),
        out_shape=jax.ShapeDtypeStruct((n, d), jnp.float32),
    )(x, scale.reshape(1, d), shift.reshape(1, d))


def _combine(x1, x2, x3, a, b):
    n, d = x1.shape

    def body(x1_ref, x2_ref, x3_ref, a_ref, b_ref, o_ref):
        o_ref[...] = (x1_ref[...] * a_ref[...] + x2_ref[...] * b_ref[...]
                      + x3_ref[...])

    sspec = pl.BlockSpec((1, 1), lambda i: (0, 0))
    bspec = pl.BlockSpec((BN, d), lambda i: (i, 0))
    return pl.pallas_call(
        body,
        grid=(n // BN,),
        in_specs=[bspec, bspec, bspec, sspec, sspec],
        out_specs=bspec,
        out_shape=jax.ShapeDtypeStruct((n, d), jnp.float32),
    )(x1, x2, x3, a.reshape(1, 1), b.reshape(1, 1))


def _autobahn(x, p_a, p_b):
    n, din = x.shape
    dout = p_a[0].shape[1]
    na = SPLIT // BN
    w2 = jnp.stack([p_a[0], p_b[0]])
    b2 = jnp.stack([p_a[1].reshape(1, dout), p_b[1].reshape(1, dout)])

    def body(x_ref, w_ref, b_ref, o_ref):
        o_ref[...] = (jnp.dot(x_ref[...], w_ref[0],
                              preferred_element_type=jnp.float32) + b_ref[0])

    def widx(i):
        s = jnp.where(i >= na, 1, 0)
        return (s, 0, 0)

    return pl.pallas_call(
        body,
        grid=(n // BN,),
        in_specs=[pl.BlockSpec((BN, din), lambda i: (i, 0)),
                  pl.BlockSpec((1, din, dout), widx),
                  pl.BlockSpec((1, 1, dout), widx)],
        out_specs=pl.BlockSpec((BN, dout), lambda i: (i, 0)),
        out_shape=jax.ShapeDtypeStruct((n, dout), jnp.float32),
    )(x, w2, b2)


def _segpool_part(y3):
    g, r, d = y3.shape
    gb = 1000

    def body(x_ref, o_ref):
        x = x_ref[...]
        o_ref[...] = jnp.broadcast_to(jnp.sum(x, axis=1, keepdims=True), x.shape)

    return pl.pallas_call(
        body,
        grid=(g // gb,),
        in_specs=[pl.BlockSpec((gb, r, d), lambda i: (i, 0, 0))],
        out_specs=pl.BlockSpec((gb, r, d), lambda i: (i, 0, 0)),
        out_shape=jax.ShapeDtypeStruct((g, r, d), jnp.float32),
    )(y3)


def _segpool_bcast(y):
    d = y.shape[1]
    p1 = _segpool_part(y[:SPLIT].reshape(10000, 5, d)).reshape(SPLIT, d)
    p2 = _segpool_part(y[SPLIT:].reshape(10000, 6, d)).reshape(C_ROWS - SPLIT, d)
    return jnp.concatenate([p1, p2], axis=0)


def _pipeline(edge_attr, cycle_attr, params, cycle_ids,
              e2c_src_1, e2c_dst_1, e2c_src_2, e2c_dst_2, c2c_src, c2c_dst,
              c2e_src_1, c2e_dst_1, c2e_src_2, c2e_dst_2, scatter):
    p = params
    one = jnp.float32(1.0)

    e2c1 = scatter(edge_attr, e2c_src_1, e2c_dst_1, C_ROWS)
    e2c2 = scatter(edge_attr, e2c_src_2, e2c_dst_2, C_ROWS)

    y2 = scatter(e2c2, c2c_src, c2c_dst, C_ROWS)
    y1 = scatter(e2c1, c2c_src, c2c_dst, C_ROWS)
    yc = scatter(cycle_attr, c2c_src, c2c_dst, C_ROWS)
    p2b = _segpool_bcast(y2)
    p1b = _segpool_bcast(y1)
    pcb = _segpool_bcast(yc)

    lift_in = jnp.concatenate([y2, p2b, y1, p1b], axis=-1)
    lift_raw, ls, lh = _mlp(p['cycle_mlp_2'], lift_in)
    lift_final = _affine_relu(lift_raw, ls, lh)

    ab1 = _autobahn(e2c1, p['ab_c5'], p['ab_c6'])
    ab2 = _autobahn(e2c2, p['ab_c5'], p['ab_c6'])
    ab_raw, as_, ah = _mlp(p['cycle_ab_mlp'], jnp.concatenate([ab1, ab2], axis=-1))
    e2c_ab = _affine_relu(ab_raw, as_, ah)

    cycle_lin = jnp.concatenate([yc, pcb], axis=-1)
    comb_c = _combine(cycle_lin, lift_final, e2c_ab,
                      one + p['eps_c1'], one + p['eps_c2'])
    cyc_raw, cs, chh = _mlp(p['cycle_mlp_1'], comb_c)
    cycle_out = _affine_relu(cyc_raw, cs, chh)

    lvl_raw, vs, vh = _mlp(p['edge_mlp_1'],
                           jnp.concatenate([lift_final, cycle_attr], axis=-1))
    lvl_final = _affine_relu(lvl_raw, vs, vh)

    ylvl = scatter(lvl_final, c2c_src, c2c_dst, C_ROWS)
    plvlb = _segpool_bcast(ylvl)
    lvl_ab = _autobahn(jnp.concatenate([ylvl, plvlb], axis=-1),
                       p['ab_l5'], p['ab_l6'])

    lv1a = scatter(ylvl, c2e_src_1, c2e_dst_1, E_ROWS)
    lv1b = scatter(plvlb, c2e_src_1, c2e_dst_1, E_ROWS)
    lv2a = scatter(ylvl, c2e_src_2, c2e_dst_2, E_ROWS)
    lv2b = scatter(plvlb, c2e_src_2, c2e_dst_2, E_ROWS)
    la1 = scatter(lvl_ab, c2e_src_1, c2e_dst_1, E_ROWS)
    la2 = scatter(lvl_ab, c2e_src_2, c2e_dst_2, E_ROWS)

    aggr_raw, gs, gh = _mlp(p['edge_mlp_3'],
                            jnp.concatenate([lv1a, lv1b, lv2a, lv2b], axis=-1))
    aggr_final = _affine_relu(aggr_raw, gs, gh)
    abt_raw, ts, th = _mlp(p['edge_mlp_4'], jnp.concatenate([la1, la2], axis=-1))
    abt_final = _affine_relu(abt_raw, ts, th)

    comb_e = _combine(edge_attr, aggr_final, abt_final,
                      one + p['eps_e1'], one + p['eps_e2'])
    edge_raw, es, eh = _mlp(p['edge_mlp_2'], comb_e)
    edge_out = _affine_relu(edge_raw, es, eh)

    return (edge_out, cycle_out)


def kernel(edge_attr, cycle_attr, params, cycle_ids,
           e2c_src_1, e2c_dst_1, e2c_src_2, e2c_dst_2, c2c_src, c2c_dst,
           c2e_src_1, c2e_dst_1, c2e_src_2, c2e_dst_2):
    return _pipeline(edge_attr, cycle_attr, params, cycle_ids,
                     e2c_src_1, e2c_dst_1, e2c_src_2, e2c_dst_2,
                     c2c_src, c2c_dst,
                     c2e_src_1, c2e_dst_1, c2e_src_2, c2e_dst_2,
                     _scatter_sc)


# speedup vs baseline: 1.0511x; 1.0511x over previous
import functools

import jax
import jax.numpy as jnp
from jax import lax
from jax.experimental import pallas as pl
from jax.experimental.pallas import tpu as pltpu
from jax.experimental.pallas import tpu_sc as plsc

E_ROWS = 100000
C_ROWS = 110000
N_CYC = 20000
SPLIT = 50000
BN_EPS = 1e-5

BN = 1000
SG = 8
_GS = {128: 128}
PAD_DI = jnp.int32(1 << 30)


@functools.lru_cache(maxsize=None)
def _make_scatter_kernel(ng, n_ch, w, ch):
    gs = _GS[w]
    gsl2 = gs.bit_length() - 1
    max_cpc = (n_ch + 1) // 2
    zch = ch // 16
    n_strips = ng // SG
    mesh = plsc.VectorSubcoreMesh(core_axis_name="c", subcore_axis_name="s")

    RB = 2048
    nrows = RB // gs

    def body(src_hbm, si_hbm, di_hbm, out_hbm,
             si2d, di2d, csi, cld2, b0, b1, accum, semg, sems):
        core = lax.axis_index("c")
        sub = lax.axis_index("s")

        bufs = (b0, b1)

        def gat(dbase, buf):
            off = pl.multiple_of(dbase & (RB - 1), gs)
            return pltpu.async_copy(src_hbm.at[csi.at[pl.ds(off, gs)]],
                                    buf, semg)

        def sca(dbase, buf):
            return pltpu.async_copy(buf, accum.at[cld2.at[(dbase >> gsl2) & (nrows - 1)]],
                                    sems, add=True)

        def burst(dbase):
            hg = gat(dbase, b0)
            hs_prev = None
            for k in range(8):
                hg.wait()
                if hs_prev is not None:
                    hs_prev.wait()
                if k < 7:
                    hg = gat(dbase + (k + 1) * gs, bufs[(k + 1) % 2])
                hs_prev = sca(dbase + k * gs, bufs[k % 2])
            hs_prev.wait()

        def pair(dbase):
            h0 = gat(dbase, b0)
            h1 = gat(dbase + gs, b1)
            h0.wait()
            pltpu.sync_copy(b0, accum.at[cld2.at[(dbase >> gsl2) & (nrows - 1)]], add=True)
            h1.wait()
            pltpu.sync_copy(b1, accum.at[cld2.at[((dbase >> gsl2) + 1) & (nrows - 1)]],
                            add=True)

        def single(dbase):
            h0 = gat(dbase, b0)
            h0.wait()
            pltpu.sync_copy(b0, accum.at[cld2.at[(dbase >> gsl2) & (nrows - 1)]], add=True)

        def chunk_step(i, _):
            c = 2 * i + core
            active = c < n_ch
            base = c * ch

            @pl.when(active)
            def _zero():
                def zrow(r, _):
                    for j in range(w // 16):
                        b0[r, pl.ds(j * 16, 16)] = jnp.zeros((16,), jnp.float32)
                    return 0
                lax.fori_loop(0, gs, zrow, 0)

                hz = [pltpu.async_copy(
                          b0, accum.at[pl.ds(sub * zch + k * gs, gs)], sems)
                      for k in range(zch // gs)]
                rem = zch % gs
                if rem:
                    hz.append(pltpu.async_copy(
                        b0.at[pl.ds(0, rem)],
                        accum.at[pl.ds(sub * zch + (zch // gs) * gs, rem)],
                        sems))
                for h in hz:
                    h.wait()

            plsc.subcore_barrier()

            @pl.when(active)
            def _accumulate():
                iot = lax.iota(jnp.int32, 16)

                def strip(t, carry):
                    cnt, dptr = carry
                    pltpu.sync_copy(si_hbm.at[sub, pl.ds(t * SG, SG)], si2d)
                    pltpu.sync_copy(di_hbm.at[sub, pl.ds(t * SG, SG)], di2d)

                    def cgrp(g, cnt):
                        for j in range(8):
                            vs = si2d[g, pl.ds(j * 16, 16)]
                            vd = di2d[g, pl.ds(j * 16, 16)]
                            m = (vd >= base) & (vd < base + ch)
                            ones = jnp.where(m, 1, 0)
                            pos = cnt + plsc.cumsum(ones) - 1
                            plsc.store_scatter(csi, [pos & (RB - 1)], vs, mask=m)
                            plsc.store_scatter(
                                cld2, [(pos >> gsl2) & (nrows - 1), pos & (gs - 1)],
                                vd - base, mask=m)
                            cnt = cnt + jnp.sum(ones)
                        return cnt
                    cnt = lax.fori_loop(0, SG, cgrp, cnt + jnp.int32(0))

                    nb = (cnt - dptr) // (8 * gs)

                    def dburst(p, _):
                        burst(dptr + p * 8 * gs)
                        return 0
                    lax.fori_loop(0, nb, dburst, 0)
                    return (cnt, dptr + nb * 8 * gs)

                cnt, dptr = lax.fori_loop(0, n_strips, strip,
                                          (jnp.int32(0), jnp.int32(0)))

                for k in range(max(1, gs // 16)):
                    p = cnt + k * 16 + iot
                    plsc.store_scatter(csi, [p & (RB - 1)],
                                       jnp.zeros((16,), jnp.int32))
                    plsc.store_scatter(cld2, [(p >> gsl2) & (nrows - 1), p & (gs - 1)],
                                       jnp.full((16,), ch, jnp.int32))

                npt = (cnt - dptr) // (2 * gs)

                def drt(p, _):
                    pair(dptr + p * 2 * gs)
                    return 0
                lax.fori_loop(0, npt, drt, 0)
                dptr2 = dptr + npt * 2 * gs

                def drs(p, _):
                    single(dptr2 + p * gs)
                    return 0
                lax.fori_loop(0, (cnt - dptr2 + gs - 1) // gs, drs, 0)

            plsc.subcore_barrier()

            @pl.when(active)
            def _flush():
                pltpu.sync_copy(accum.at[pl.ds(sub * zch, zch)],
                                out_hbm.at[pl.ds(base + sub * zch, zch)])

            plsc.subcore_barrier()
            return 0

        lax.fori_loop(0, max_cpc, chunk_step, 0)

    return pl.kernel(
        body,
        out_type=jax.ShapeDtypeStruct((n_ch * ch, w), jnp.float32),
        mesh=mesh,
        compiler_params=pltpu.CompilerParams(needs_layout_passes=False),
        scratch_types=[
            pltpu.VMEM((SG, 128), jnp.int32),
            pltpu.VMEM((SG, 128), jnp.int32),
            pltpu.VMEM((RB,), jnp.int32),
            pltpu.VMEM((nrows, gs), jnp.int32),
            pltpu.VMEM((gs, w), jnp.float32),
            pltpu.VMEM((gs, w), jnp.float32),
            pltpu.VMEM_SHARED((ch + 8, w), jnp.float32),
            pltpu.SemaphoreType.DMA,
            pltpu.SemaphoreType.DMA,
        ],
    )


def _scatter_sc(src, si, di, n_out):
    m = si.shape[0]
    w = src.shape[1]
    mult = 16 * SG * 128
    m_pad = -(-m // mult) * mult
    if m_pad != m:
        si = jnp.concatenate([si, jnp.zeros((m_pad - m,), jnp.int32)])
        di = jnp.concatenate([di, jnp.full((m_pad - m,), PAD_DI, jnp.int32)])
    ng = m_pad // 16 // 128
    ch = 11264 if n_out > 102400 else 10240
    n_ch = -(-n_out // ch)
    si3 = si.reshape(16, ng, 128)
    di3 = di.reshape(16, ng, 128)
    k = _make_scatter_kernel(ng, n_ch, w, ch)
    out = k(src, si3, di3)
    return out[:n_out]


def _linear_bn(x, w, b, scale=None, shift=None):
    n, din = x.shape
    dout = w.shape[1]
    nb = n // BN
    apply_act = scale is not None

    def body(*refs):
        if apply_act:
            x_ref, s_ref, h_ref, w_ref, b_ref, y_ref, st_ref = refs
            xx = jnp.maximum(x_ref[...] * s_ref[...] + h_ref[...], 0.0)
        else:
            x_ref, w_ref, b_ref, y_ref, st_ref = refs
            xx = x_ref[...]
        y = jnp.dot(xx, w_ref[...], preferred_element_type=jnp.float32) + b_ref[...]
        y_ref[...] = y
        st = jnp.stack([jnp.sum(y, axis=0), jnp.sum(y * y, axis=0)], axis=0)

        @pl.when(pl.program_id(0) == 0)
        def _():
            st_ref[...] = st

        @pl.when(pl.program_id(0) != 0)
        def _():
            st_ref[...] += st

    in_specs = [pl.BlockSpec((BN, din), lambda i: (i, 0))]
    args = [x]
    if apply_act:
        in_specs += [pl.BlockSpec((1, din), lambda i: (0, 0)),
                     pl.BlockSpec((1, din), lambda i: (0, 0))]
        args += [scale.reshape(1, din), shift.reshape(1, din)]
    in_specs += [pl.BlockSpec((din, dout), lambda i: (0, 0)),
                 pl.BlockSpec((1, dout), lambda i: (0, 0))]
    args += [w, b.reshape(1, dout)]

    y, st = pl.pallas_call(
        body,
        grid=(nb,),
        in_specs=in_specs,
        out_specs=[pl.BlockSpec((BN, dout), lambda i: (i, 0)),
                   pl.BlockSpec((2, dout), lambda i: (0, 0))],
        out_shape=[jax.ShapeDtypeStruct((n, dout), jnp.float32),
                   jax.ShapeDtypeStruct((2, dout), jnp.float32)],
    )(*args)
    return y, st


def _bn_coeffs(st, n, g, be):
    mu = st[0] / n
    var = st[1] / n - mu * mu
    inv = g * lax.rsqrt(var + BN_EPS)
    return inv, be - mu * inv


def _mlp(params, x):
    n = x.shape[0]
    scale = shift = None
    for (w, b, g, be) in params:
        x, st = _linear_bn(x, w, b, scale, shift)
        scale, shift = _bn_coeffs(st, n, g, be)
    return x, scale, shift


def _affine_relu(x, scale, shift):
    n, d = x.shape

    def body(x_ref, s_ref, h_ref, o_ref):
        o_ref[...] = jnp.maximum(x_ref[...] * s_ref[...] + h_ref[...], 0.0)

    return pl.pallas_call(
        body,
        grid=(n // BN,),
        in_specs=[pl.BlockSpec((BN, d), lambda i: (i, 0)),
                  pl.BlockSpec((1, d), lambda i: (0, 0)),
                  pl.BlockSpec((1, d), lambda i: (0, 0))],
        out_specs=pl.BlockSpec((BN, d), lambda i: (i, 0)),
        out_shape=jax.ShapeDtypeStruct((n, d), jnp.float32),
    )(x, scale.reshape(1, d), shift.reshape(1, d))


def _combine(x1, x2, x3, a, b):
    n, d = x1.shape

    def body(x1_ref, x2_ref, x3_ref, a_ref, b_ref, o_ref):
        o_ref[...] = (x1_ref[...] * a_ref[...] + x2_ref[...] * b_ref[...]
                      + x3_ref[...])

    sspec = pl.BlockSpec((1, 1), lambda i: (0, 0))
    bspec = pl.BlockSpec((BN, d), lambda i: (i, 0))
    return pl.pallas_call(
        body,
        grid=(n // BN,),
        in_specs=[bspec, bspec, bspec, sspec, sspec],
        out_specs=bspec,
        out_shape=jax.ShapeDtypeStruct((n, d), jnp.float32),
    )(x1, x2, x3, a.reshape(1, 1), b.reshape(1, 1))


def _autobahn(x, p_a, p_b):
    n, din = x.shape
    dout = p_a[0].shape[1]
    na = SPLIT // BN
    w2 = jnp.stack([p_a[0], p_b[0]])
    b2 = jnp.stack([p_a[1].reshape(1, dout), p_b[1].reshape(1, dout)])

    def body(x_ref, w_ref, b_ref, o_ref):
        o_ref[...] = (jnp.dot(x_ref[...], w_ref[0],
                              preferred_element_type=jnp.float32) + b_ref[0])

    def widx(i):
        s = jnp.where(i >= na, 1, 0)
        return (s, 0, 0)

    return pl.pallas_call(
        body,
        grid=(n // BN,),
        in_specs=[pl.BlockSpec((BN, din), lambda i: (i, 0)),
                  pl.BlockSpec((1, din, dout), widx),
                  pl.BlockSpec((1, 1, dout), widx)],
        out_specs=pl.BlockSpec((BN, dout), lambda i: (i, 0)),
        out_shape=jax.ShapeDtypeStruct((n, dout), jnp.float32),
    )(x, w2, b2)


def _segpool_part(y3):
    g, r, d = y3.shape
    gb = 1000

    def body(x_ref, o_ref):
        x = x_ref[...]
        o_ref[...] = jnp.broadcast_to(jnp.sum(x, axis=1, keepdims=True), x.shape)

    return pl.pallas_call(
        body,
        grid=(g // gb,),
        in_specs=[pl.BlockSpec((gb, r, d), lambda i: (i, 0, 0))],
        out_specs=pl.BlockSpec((gb, r, d), lambda i: (i, 0, 0)),
        out_shape=jax.ShapeDtypeStruct((g, r, d), jnp.float32),
    )(y3)


def _segpool_bcast(y):
    d = y.shape[1]
    p1 = _segpool_part(y[:SPLIT].reshape(10000, 5, d)).reshape(SPLIT, d)
    p2 = _segpool_part(y[SPLIT:].reshape(10000, 6, d)).reshape(C_ROWS - SPLIT, d)
    return jnp.concatenate([p1, p2], axis=0)


def _pipeline(edge_attr, cycle_attr, params, cycle_ids,
              e2c_src_1, e2c_dst_1, e2c_src_2, e2c_dst_2, c2c_src, c2c_dst,
              c2e_src_1, c2e_dst_1, c2e_src_2, c2e_dst_2, scatter):
    p = params
    one = jnp.float32(1.0)

    e2c1 = scatter(edge_attr, e2c_src_1, e2c_dst_1, C_ROWS)
    e2c2 = scatter(edge_attr, e2c_src_2, e2c_dst_2, C_ROWS)

    y2 = scatter(e2c2, c2c_src, c2c_dst, C_ROWS)
    y1 = scatter(e2c1, c2c_src, c2c_dst, C_ROWS)
    yc = scatter(cycle_attr, c2c_src, c2c_dst, C_ROWS)
    p2b = _segpool_bcast(y2)
    p1b = _segpool_bcast(y1)
    pcb = _segpool_bcast(yc)

    lift_in = jnp.concatenate([y2, p2b, y1, p1b], axis=-1)
    lift_raw, ls, lh = _mlp(p['cycle_mlp_2'], lift_in)
    lift_final = _affine_relu(lift_raw, ls, lh)

    ab1 = _autobahn(e2c1, p['ab_c5'], p['ab_c6'])
    ab2 = _autobahn(e2c2, p['ab_c5'], p['ab_c6'])
    ab_raw, as_, ah = _mlp(p['cycle_ab_mlp'], jnp.concatenate([ab1, ab2], axis=-1))
    e2c_ab = _affine_relu(ab_raw, as_, ah)

    cycle_lin = jnp.concatenate([yc, pcb], axis=-1)
    comb_c = _combine(cycle_lin, lift_final, e2c_ab,
                      one + p['eps_c1'], one + p['eps_c2'])
    cyc_raw, cs, chh = _mlp(p['cycle_mlp_1'], comb_c)
    cycle_out = _affine_relu(cyc_raw, cs, chh)

    lvl_raw, vs, vh = _mlp(p['edge_mlp_1'],
                           jnp.concatenate([lift_final, cycle_attr], axis=-1))
    lvl_final = _affine_relu(lvl_raw, vs, vh)

    ylvl = scatter(lvl_final, c2c_src, c2c_dst, C_ROWS)
    plvlb = _segpool_bcast(ylvl)
    lvl_ab = _autobahn(jnp.concatenate([ylvl, plvlb], axis=-1),
                       p['ab_l5'], p['ab_l6'])

    lv1a = scatter(ylvl, c2e_src_1, c2e_dst_1, E_ROWS)
    lv1b = scatter(plvlb, c2e_src_1, c2e_dst_1, E_ROWS)
    lv2a = scatter(ylvl, c2e_src_2, c2e_dst_2, E_ROWS)
    lv2b = scatter(plvlb, c2e_src_2, c2e_dst_2, E_ROWS)
    la1 = scatter(lvl_ab, c2e_src_1, c2e_dst_1, E_ROWS)
    la2 = scatter(lvl_ab, c2e_src_2, c2e_dst_2, E_ROWS)

    aggr_raw, gsc, gh = _mlp(p['edge_mlp_3'],
                             jnp.concatenate([lv1a, lv1b, lv2a, lv2b], axis=-1))
    aggr_final = _affine_relu(aggr_raw, gsc, gh)
    abt_raw, ts, th = _mlp(p['edge_mlp_4'],
                           jnp.concatenate([la1, la2], axis=-1))
    abt_final = _affine_relu(abt_raw, ts, th)

    comb_e = _combine(edge_attr, aggr_final, abt_final,
                      one + p['eps_e1'], one + p['eps_e2'])
    edge_raw, es, eh = _mlp(p['edge_mlp_2'], comb_e)
    edge_out = _affine_relu(edge_raw, es, eh)

    return (edge_out, cycle_out)


def kernel(edge_attr, cycle_attr, params, cycle_ids,
           e2c_src_1, e2c_dst_1, e2c_src_2, e2c_dst_2, c2c_src, c2c_dst,
           c2e_src_1, c2e_dst_1, c2e_src_2, c2e_dst_2):
    return _pipeline(edge_attr, cycle_attr, params, cycle_ids,
                     e2c_src_1, e2c_dst_1, e2c_src_2, e2c_dst_2,
                     c2c_src, c2c_dst,
                     c2e_src_1, c2e_dst_1, c2e_src_2, c2e_dst_2,
                     _scatter_sc)


# speedup vs baseline: 1.0945x; 1.0412x over previous
import functools

import jax
import jax.numpy as jnp
from jax import lax
from jax.experimental import pallas as pl
from jax.experimental.pallas import tpu as pltpu
from jax.experimental.pallas import tpu_sc as plsc

E_ROWS = 100000
C_ROWS = 110000
N_CYC = 20000
SPLIT = 50000
BN_EPS = 1e-5

BN = 1000
SG = 8
_GS = {128: 128}
PAD_DI = jnp.int32(1 << 30)


@functools.lru_cache(maxsize=None)
def _make_scatter_kernel(ng, n_ch, w, ch):
    gs = _GS[w]
    gsl2 = gs.bit_length() - 1
    max_cpc = (n_ch + 1) // 2
    zch = ch // 16
    n_strips = ng // SG
    mesh = plsc.VectorSubcoreMesh(core_axis_name="c", subcore_axis_name="s")

    RB = 2048
    nrows = RB // gs

    def body(src_hbm, si_hbm, di_hbm, out_hbm,
             si2d, di2d, csi, cld2, b0, b1, accum, semg, sems, semi):
        core = lax.axis_index("c")
        sub = lax.axis_index("s")

        bufs = (b0, b1)

        def gat(dbase, buf):
            off = pl.multiple_of(dbase & (RB - 1), gs)
            return pltpu.async_copy(src_hbm.at[csi.at[pl.ds(off, gs)]],
                                    buf, semg)

        def sca(dbase, buf):
            return pltpu.async_copy(buf, accum.at[cld2.at[(dbase >> gsl2) & (nrows - 1)]],
                                    sems, add=True)

        def burst(dbase):
            hg = gat(dbase, b0)
            hs_prev = None
            for k in range(8):
                hg.wait()
                if hs_prev is not None:
                    hs_prev.wait()
                if k < 7:
                    hg = gat(dbase + (k + 1) * gs, bufs[(k + 1) % 2])
                hs_prev = sca(dbase + k * gs, bufs[k % 2])
            hs_prev.wait()

        def pair(dbase):
            h0 = gat(dbase, b0)
            h1 = gat(dbase + gs, b1)
            h0.wait()
            pltpu.sync_copy(b0, accum.at[cld2.at[(dbase >> gsl2) & (nrows - 1)]], add=True)
            h1.wait()
            pltpu.sync_copy(b1, accum.at[cld2.at[((dbase >> gsl2) + 1) & (nrows - 1)]],
                            add=True)

        def single(dbase):
            h0 = gat(dbase, b0)
            h0.wait()
            pltpu.sync_copy(b0, accum.at[cld2.at[(dbase >> gsl2) & (nrows - 1)]], add=True)

        def chunk_step(i, _):
            c = 2 * i + core
            active = c < n_ch
            base = c * ch

            @pl.when(active)
            def _zero():
                def zrow(r, _):
                    for j in range(w // 16):
                        b0[r, pl.ds(j * 16, 16)] = jnp.zeros((16,), jnp.float32)
                    return 0
                lax.fori_loop(0, gs, zrow, 0)

                hz = [pltpu.async_copy(
                          b0, accum.at[pl.ds(sub * zch + k * gs, gs)], sems)
                      for k in range(zch // gs)]
                rem = zch % gs
                if rem:
                    hz.append(pltpu.async_copy(
                        b0.at[pl.ds(0, rem)],
                        accum.at[pl.ds(sub * zch + (zch // gs) * gs, rem)],
                        sems))
                for h in hz:
                    h.wait()

            plsc.subcore_barrier()

            @pl.when(active)
            def _accumulate():
                iot = lax.iota(jnp.int32, 16)

                pltpu.async_copy(si_hbm.at[sub, pl.ds(0, SG)], si2d.at[0], semi)
                pltpu.async_copy(di_hbm.at[sub, pl.ds(0, SG)], di2d.at[0], semi)

                def strip(t, carry):
                    cnt, dptr = carry
                    cur = t & 1
                    pltpu.make_async_copy(si_hbm.at[sub, pl.ds(t * SG, SG)],
                                          si2d.at[cur], semi).wait()
                    pltpu.make_async_copy(di_hbm.at[sub, pl.ds(t * SG, SG)],
                                          di2d.at[cur], semi).wait()

                    @pl.when(t + 1 < n_strips)
                    def _prefetch():
                        nxt = (t + 1) & 1
                        pltpu.async_copy(si_hbm.at[sub, pl.ds((t + 1) * SG, SG)],
                                         si2d.at[nxt], semi)
                        pltpu.async_copy(di_hbm.at[sub, pl.ds((t + 1) * SG, SG)],
                                         di2d.at[nxt], semi)

                    def cgrp(g, cnt):
                        for j in range(8):
                            vs = si2d[cur, g, pl.ds(j * 16, 16)]
                            vd = di2d[cur, g, pl.ds(j * 16, 16)]
                            m = (vd >= base) & (vd < base + ch)
                            ones = jnp.where(m, 1, 0)
                            pos = cnt + plsc.cumsum(ones) - 1
                            plsc.store_scatter(csi, [pos & (RB - 1)], vs, mask=m)
                            plsc.store_scatter(
                                cld2, [(pos >> gsl2) & (nrows - 1), pos & (gs - 1)],
                                vd - base, mask=m)
                            cnt = cnt + jnp.sum(ones)
                        return cnt
                    cnt = lax.fori_loop(0, SG, cgrp, cnt + jnp.int32(0))

                    nb = (cnt - dptr) // (8 * gs)

                    def dburst(p, _):
                        burst(dptr + p * 8 * gs)
                        return 0
                    lax.fori_loop(0, nb, dburst, 0)
                    return (cnt, dptr + nb * 8 * gs)

                cnt, dptr = lax.fori_loop(0, n_strips, strip,
                                          (jnp.int32(0), jnp.int32(0)))

                for k in range(max(1, gs // 16)):
                    p = cnt + k * 16 + iot
                    plsc.store_scatter(csi, [p & (RB - 1)],
                                       jnp.zeros((16,), jnp.int32))
                    plsc.store_scatter(cld2, [(p >> gsl2) & (nrows - 1), p & (gs - 1)],
                                       jnp.full((16,), ch, jnp.int32))

                npt = (cnt - dptr) // (2 * gs)

                def drt(p, _):
                    pair(dptr + p * 2 * gs)
                    return 0
                lax.fori_loop(0, npt, drt, 0)
                dptr2 = dptr + npt * 2 * gs

                def drs(p, _):
                    single(dptr2 + p * gs)
                    return 0
                lax.fori_loop(0, (cnt - dptr2 + gs - 1) // gs, drs, 0)

            plsc.subcore_barrier()

            @pl.when(active)
            def _flush():
                pltpu.sync_copy(accum.at[pl.ds(sub * zch, zch)],
                                out_hbm.at[pl.ds(base + sub * zch, zch)])

            plsc.subcore_barrier()
            return 0

        lax.fori_loop(0, max_cpc, chunk_step, 0)

    return pl.kernel(
        body,
        out_type=jax.ShapeDtypeStruct((n_ch * ch, w), jnp.float32),
        mesh=mesh,
        compiler_params=pltpu.CompilerParams(needs_layout_passes=False),
        scratch_types=[
            pltpu.VMEM((2, SG, 128), jnp.int32),
            pltpu.VMEM((2, SG, 128), jnp.int32),
            pltpu.VMEM((RB,), jnp.int32),
            pltpu.VMEM((nrows, gs), jnp.int32),
            pltpu.VMEM((gs, w), jnp.float32),
            pltpu.VMEM((gs, w), jnp.float32),
            pltpu.VMEM_SHARED((ch + 8, w), jnp.float32),
            pltpu.SemaphoreType.DMA,
            pltpu.SemaphoreType.DMA,
            pltpu.SemaphoreType.DMA,
        ],
    )


def _scatter_sc(src, si, di, n_out):
    m = si.shape[0]
    w = src.shape[1]
    mult = 16 * SG * 128
    m_pad = -(-m // mult) * mult
    if m_pad != m:
        si = jnp.concatenate([si, jnp.zeros((m_pad - m,), jnp.int32)])
        di = jnp.concatenate([di, jnp.full((m_pad - m,), PAD_DI, jnp.int32)])
    ng = m_pad // 16 // 128
    ch = 11136 if n_out > 102400 else 10240
    n_ch = -(-n_out // ch)
    si3 = si.reshape(16, ng, 128)
    di3 = di.reshape(16, ng, 128)
    k = _make_scatter_kernel(ng, n_ch, w, ch)
    out = k(src, si3, di3)
    return out[:n_out]


def _linear_bn(x, w, b, scale=None, shift=None):
    n, din = x.shape
    dout = w.shape[1]
    nb = n // BN
    apply_act = scale is not None

    def body(*refs):
        if apply_act:
            x_ref, s_ref, h_ref, w_ref, b_ref, y_ref, st_ref = refs
            xx = jnp.maximum(x_ref[...] * s_ref[...] + h_ref[...], 0.0)
        else:
            x_ref, w_ref, b_ref, y_ref, st_ref = refs
            xx = x_ref[...]
        y = jnp.dot(xx, w_ref[...], preferred_element_type=jnp.float32) + b_ref[...]
        y_ref[...] = y
        st = jnp.stack([jnp.sum(y, axis=0), jnp.sum(y * y, axis=0)], axis=0)

        @pl.when(pl.program_id(0) == 0)
        def _():
            st_ref[...] = st

        @pl.when(pl.program_id(0) != 0)
        def _():
            st_ref[...] += st

    in_specs = [pl.BlockSpec((BN, din), lambda i: (i, 0))]
    args = [x]
    if apply_act:
        in_specs += [pl.BlockSpec((1, din), lambda i: (0, 0)),
                     pl.BlockSpec((1, din), lambda i: (0, 0))]
        args += [scale.reshape(1, din), shift.reshape(1, din)]
    in_specs += [pl.BlockSpec((din, dout), lambda i: (0, 0)),
                 pl.BlockSpec((1, dout), lambda i: (0, 0))]
    args += [w, b.reshape(1, dout)]

    y, st = pl.pallas_call(
        body,
        grid=(nb,),
        in_specs=in_specs,
        out_specs=[pl.BlockSpec((BN, dout), lambda i: (i, 0)),
                   pl.BlockSpec((2, dout), lambda i: (0, 0))],
        out_shape=[jax.ShapeDtypeStruct((n, dout), jnp.float32),
                   jax.ShapeDtypeStruct((2, dout), jnp.float32)],
    )(*args)
    return y, st


def _bn_coeffs(st, n, g, be):
    mu = st[0] / n
    var = st[1] / n - mu * mu
    inv = g * lax.rsqrt(var + BN_EPS)
    return inv, be - mu * inv


def _mlp(params, x):
    n = x.shape[0]
    scale = shift = None
    for (w, b, g, be) in params:
        x, st = _linear_bn(x, w, b, scale, shift)
        scale, shift = _bn_coeffs(st, n, g, be)
    return x, scale, shift


def _affine_relu(x, scale, shift):
    n, d = x.shape

    def body(x_ref, s_ref, h_ref, o_ref):
        o_ref[...] = jnp.maximum(x_ref[...] * s_ref[...] + h_ref[...], 0.0)

    return pl.pallas_call(
        body,
        grid=(n // BN,),
        in_specs=[pl.BlockSpec((BN, d), lambda i: (i, 0)),
                  pl.BlockSpec((1, d), lambda i: (0, 0)),
                  pl.BlockSpec((1, d), lambda i: (0, 0))],
        out_specs=pl.BlockSpec((BN, d), lambda i: (i, 0)),
        out_shape=jax.ShapeDtypeStruct((n, d), jnp.float32),
    )(x, scale.reshape(1, d), shift.reshape(1, d))


def _combine(x1, x2, x3, a, b):
    n, d = x1.shape

    def body(x1_ref, x2_ref, x3_ref, a_ref, b_ref, o_ref):
        o_ref[...] = (x1_ref[...] * a_ref[...] + x2_ref[...] * b_ref[...]
                      + x3_ref[...])

    sspec = pl.BlockSpec((1, 1), lambda i: (0, 0))
    bspec = pl.BlockSpec((BN, d), lambda i: (i, 0))
    return pl.pallas_call(
        body,
        grid=(n // BN,),
        in_specs=[bspec, bspec, bspec, sspec, sspec],
        out_specs=bspec,
        out_shape=jax.ShapeDtypeStruct((n, d), jnp.float32),
    )(x1, x2, x3, a.reshape(1, 1), b.reshape(1, 1))


def _autobahn(x, p_a, p_b):
    n, din = x.shape
    dout = p_a[0].shape[1]
    na = SPLIT // BN
    w2 = jnp.stack([p_a[0], p_b[0]])
    b2 = jnp.stack([p_a[1].reshape(1, dout), p_b[1].reshape(1, dout)])

    def body(x_ref, w_ref, b_ref, o_ref):
        o_ref[...] = (jnp.dot(x_ref[...], w_ref[0],
                              preferred_element_type=jnp.float32) + b_ref[0])

    def widx(i):
        s = jnp.where(i >= na, 1, 0)
        return (s, 0, 0)

    return pl.pallas_call(
        body,
        grid=(n // BN,),
        in_specs=[pl.BlockSpec((BN, din), lambda i: (i, 0)),
                  pl.BlockSpec((1, din, dout), widx),
                  pl.BlockSpec((1, 1, dout), widx)],
        out_specs=pl.BlockSpec((BN, dout), lambda i: (i, 0)),
        out_shape=jax.ShapeDtypeStruct((n, dout), jnp.float32),
    )(x, w2, b2)


def _segpool_part(y3):
    g, r, d = y3.shape
    gb = 1000

    def body(x_ref, o_ref):
        x = x_ref[...]
        o_ref[...] = jnp.broadcast_to(jnp.sum(x, axis=1, keepdims=True), x.shape)

    return pl.pallas_call(
        body,
        grid=(g // gb,),
        in_specs=[pl.BlockSpec((gb, r, d), lambda i: (i, 0, 0))],
        out_specs=pl.BlockSpec((gb, r, d), lambda i: (i, 0, 0)),
        out_shape=jax.ShapeDtypeStruct((g, r, d), jnp.float32),
    )(y3)


def _segpool_bcast(y):
    d = y.shape[1]
    p1 = _segpool_part(y[:SPLIT].reshape(10000, 5, d)).reshape(SPLIT, d)
    p2 = _segpool_part(y[SPLIT:].reshape(10000, 6, d)).reshape(C_ROWS - SPLIT, d)
    return jnp.concatenate([p1, p2], axis=0)


def _pipeline(edge_attr, cycle_attr, params, cycle_ids,
              e2c_src_1, e2c_dst_1, e2c_src_2, e2c_dst_2, c2c_src, c2c_dst,
              c2e_src_1, c2e_dst_1, c2e_src_2, c2e_dst_2, scatter):
    p = params
    one = jnp.float32(1.0)

    e2c1 = scatter(edge_attr, e2c_src_1, e2c_dst_1, C_ROWS)
    e2c2 = scatter(edge_attr, e2c_src_2, e2c_dst_2, C_ROWS)

    y2 = scatter(e2c2, c2c_src, c2c_dst, C_ROWS)
    y1 = scatter(e2c1, c2c_src, c2c_dst, C_ROWS)
    yc = scatter(cycle_attr, c2c_src, c2c_dst, C_ROWS)
    p2b = _segpool_bcast(y2)
    p1b = _segpool_bcast(y1)
    pcb = _segpool_bcast(yc)

    lift_in = jnp.concatenate([y2, p2b, y1, p1b], axis=-1)
    lift_raw, ls, lh = _mlp(p['cycle_mlp_2'], lift_in)
    lift_final = _affine_relu(lift_raw, ls, lh)

    ab1 = _autobahn(e2c1, p['ab_c5'], p['ab_c6'])
    ab2 = _autobahn(e2c2, p['ab_c5'], p['ab_c6'])
    ab_raw, as_, ah = _mlp(p['cycle_ab_mlp'], jnp.concatenate([ab1, ab2], axis=-1))
    e2c_ab = _affine_relu(ab_raw, as_, ah)

    cycle_lin = jnp.concatenate([yc, pcb], axis=-1)
    comb_c = _combine(cycle_lin, lift_final, e2c_ab,
                      one + p['eps_c1'], one + p['eps_c2'])
    cyc_raw, cs, chh = _mlp(p['cycle_mlp_1'], comb_c)
    cycle_out = _affine_relu(cyc_raw, cs, chh)

    lvl_raw, vs, vh = _mlp(p['edge_mlp_1'],
                           jnp.concatenate([lift_final, cycle_attr], axis=-1))
    lvl_final = _affine_relu(lvl_raw, vs, vh)

    ylvl = scatter(lvl_final, c2c_src, c2c_dst, C_ROWS)
    plvlb = _segpool_bcast(ylvl)
    lvl_ab = _autobahn(jnp.concatenate([ylvl, plvlb], axis=-1),
                       p['ab_l5'], p['ab_l6'])

    lv1a = scatter(ylvl, c2e_src_1, c2e_dst_1, E_ROWS)
    lv1b = scatter(plvlb, c2e_src_1, c2e_dst_1, E_ROWS)
    lv2a = scatter(ylvl, c2e_src_2, c2e_dst_2, E_ROWS)
    lv2b = scatter(plvlb, c2e_src_2, c2e_dst_2, E_ROWS)
    la1 = scatter(lvl_ab, c2e_src_1, c2e_dst_1, E_ROWS)
    la2 = scatter(lvl_ab, c2e_src_2, c2e_dst_2, E_ROWS)

    aggr_raw, gsc, gh = _mlp(p['edge_mlp_3'],
                             jnp.concatenate([lv1a, lv1b, lv2a, lv2b], axis=-1))
    aggr_final = _affine_relu(aggr_raw, gsc, gh)
    abt_raw, ts, th = _mlp(p['edge_mlp_4'],
                           jnp.concatenate([la1, la2], axis=-1))
    abt_final = _affine_relu(abt_raw, ts, th)

    comb_e = _combine(edge_attr, aggr_final, abt_final,
                      one + p['eps_e1'], one + p['eps_e2'])
    edge_raw, es, eh = _mlp(p['edge_mlp_2'], comb_e)
    edge_out = _affine_relu(edge_raw, es, eh)

    return (edge_out, cycle_out)


def kernel(edge_attr, cycle_attr, params, cycle_ids,
           e2c_src_1, e2c_dst_1, e2c_src_2, e2c_dst_2, c2c_src, c2c_dst,
           c2e_src_1, c2e_dst_1, c2e_src_2, c2e_dst_2):
    return _pipeline(edge_attr, cycle_attr, params, cycle_ids,
                     e2c_src_1, e2c_dst_1, e2c_src_2, e2c_dst_2,
                     c2c_src, c2c_dst,
                     c2e_src_1, c2e_dst_1, c2e_src_2, c2e_dst_2,
                     _scatter_sc)


# speedup vs baseline: 1.1067x; 1.0111x over previous
import functools

import jax
import jax.numpy as jnp
from jax import lax
from jax.experimental import pallas as pl
from jax.experimental.pallas import tpu as pltpu
from jax.experimental.pallas import tpu_sc as plsc

E_ROWS = 100000
C_ROWS = 110000
N_CYC = 20000
SPLIT = 50000
BN_EPS = 1e-5

BN = 1000
SG = 8
_GS = {128: 128}
PAD_DI = jnp.int32(1 << 30)


@functools.lru_cache(maxsize=None)
def _make_scatter_kernel(ng, n_ch, w, ch):
    gs = _GS[w]
    gsl2 = gs.bit_length() - 1
    max_cpc = (n_ch + 1) // 2
    zch = ch // 16
    n_strips = ng // SG
    mesh = plsc.VectorSubcoreMesh(core_axis_name="c", subcore_axis_name="s")

    RB = 2048
    nrows = RB // gs

    def body(src_hbm, si_hbm, di_hbm, out_hbm,
             si2d, di2d, csi, cld2, b0, b1, accum, semg, sems, semi):
        core = lax.axis_index("c")
        sub = lax.axis_index("s")

        bufs = (b0, b1)

        def gat(dbase, buf):
            off = pl.multiple_of(dbase & (RB - 1), gs)
            return pltpu.async_copy(src_hbm.at[csi.at[pl.ds(off, gs)]],
                                    buf, semg)

        def sca(dbase, buf):
            return pltpu.async_copy(buf, accum.at[cld2.at[(dbase >> gsl2) & (nrows - 1)]],
                                    sems, add=True)

        def burst(dbase):
            hg = gat(dbase, b0)
            hs_prev = None
            for k in range(8):
                hg.wait()
                if hs_prev is not None:
                    hs_prev.wait()
                if k < 7:
                    hg = gat(dbase + (k + 1) * gs, bufs[(k + 1) % 2])
                hs_prev = sca(dbase + k * gs, bufs[k % 2])
            hs_prev.wait()

        def pair(dbase):
            h0 = gat(dbase, b0)
            h1 = gat(dbase + gs, b1)
            h0.wait()
            pltpu.sync_copy(b0, accum.at[cld2.at[(dbase >> gsl2) & (nrows - 1)]], add=True)
            h1.wait()
            pltpu.sync_copy(b1, accum.at[cld2.at[((dbase >> gsl2) + 1) & (nrows - 1)]],
                            add=True)

        def single(dbase):
            h0 = gat(dbase, b0)
            h0.wait()
            pltpu.sync_copy(b0, accum.at[cld2.at[(dbase >> gsl2) & (nrows - 1)]], add=True)

        def chunk_step(i, _):
            c = 2 * i + core
            active = c < n_ch
            base = c * ch

            @pl.when(active)
            def _zero():
                def zrow(r, _):
                    for j in range(w // 16):
                        b0[r, pl.ds(j * 16, 16)] = jnp.zeros((16,), jnp.float32)
                    return 0
                lax.fori_loop(0, gs, zrow, 0)

                hz = [pltpu.async_copy(
                          b0, accum.at[pl.ds(sub * zch + k * gs, gs)], sems)
                      for k in range(zch // gs)]
                rem = zch % gs
                if rem:
                    hz.append(pltpu.async_copy(
                        b0.at[pl.ds(0, rem)],
                        accum.at[pl.ds(sub * zch + (zch // gs) * gs, rem)],
                        sems))
                for h in hz:
                    h.wait()

            plsc.subcore_barrier()

            @pl.when(active)
            def _accumulate():
                iot = lax.iota(jnp.int32, 16)

                pltpu.async_copy(si_hbm.at[sub, pl.ds(0, SG)], si2d.at[0], semi)
                pltpu.async_copy(di_hbm.at[sub, pl.ds(0, SG)], di2d.at[0], semi)

                def strip(t, carry):
                    cnt, dptr = carry
                    cur = t & 1
                    pltpu.make_async_copy(si_hbm.at[sub, pl.ds(t * SG, SG)],
                                          si2d.at[cur], semi).wait()
                    pltpu.make_async_copy(di_hbm.at[sub, pl.ds(t * SG, SG)],
                                          di2d.at[cur], semi).wait()

                    @pl.when(t + 1 < n_strips)
                    def _prefetch():
                        nxt = (t + 1) & 1
                        pltpu.async_copy(si_hbm.at[sub, pl.ds((t + 1) * SG, SG)],
                                         si2d.at[nxt], semi)
                        pltpu.async_copy(di_hbm.at[sub, pl.ds((t + 1) * SG, SG)],
                                         di2d.at[nxt], semi)

                    def cgrp(g, cnt):
                        lanes = []
                        for j in range(8):
                            vs = si2d[cur, g, pl.ds(j * 16, 16)]
                            vd = di2d[cur, g, pl.ds(j * 16, 16)]
                            m = (vd >= base) & (vd < base + ch)
                            ones = jnp.where(m, 1, 0)
                            lanes.append((vs, vd, m, ones, jnp.sum(ones)))
                        offs = []
                        for (_, _, _, _, s) in lanes:
                            offs.append(cnt)
                            cnt = cnt + s
                        for (vs, vd, m, ones, _), off in zip(lanes, offs):
                            pos = off + plsc.cumsum(ones) - 1
                            plsc.store_scatter(csi, [pos & (RB - 1)], vs, mask=m)
                            plsc.store_scatter(
                                cld2, [(pos >> gsl2) & (nrows - 1), pos & (gs - 1)],
                                vd - base, mask=m)
                        return cnt
                    cnt = lax.fori_loop(0, SG, cgrp, cnt + jnp.int32(0))

                    nb = (cnt - dptr) // (8 * gs)

                    def dburst(p, _):
                        burst(dptr + p * 8 * gs)
                        return 0
                    lax.fori_loop(0, nb, dburst, 0)
                    return (cnt, dptr + nb * 8 * gs)

                cnt, dptr = lax.fori_loop(0, n_strips, strip,
                                          (jnp.int32(0), jnp.int32(0)))

                for k in range(max(1, gs // 16)):
                    p = cnt + k * 16 + iot
                    plsc.store_scatter(csi, [p & (RB - 1)],
                                       jnp.zeros((16,), jnp.int32))
                    plsc.store_scatter(cld2, [(p >> gsl2) & (nrows - 1), p & (gs - 1)],
                                       jnp.full((16,), ch, jnp.int32))

                npt = (cnt - dptr) // (2 * gs)

                def drt(p, _):
                    pair(dptr + p * 2 * gs)
                    return 0
                lax.fori_loop(0, npt, drt, 0)
                dptr2 = dptr + npt * 2 * gs

                def drs(p, _):
                    single(dptr2 + p * gs)
                    return 0
                lax.fori_loop(0, (cnt - dptr2 + gs - 1) // gs, drs, 0)

            plsc.subcore_barrier()

            @pl.when(active)
            def _flush():
                pltpu.sync_copy(accum.at[pl.ds(sub * zch, zch)],
                                out_hbm.at[pl.ds(base + sub * zch, zch)])

            plsc.subcore_barrier()
            return 0

        lax.fori_loop(0, max_cpc, chunk_step, 0)

    return pl.kernel(
        body,
        out_type=jax.ShapeDtypeStruct((n_ch * ch, w), jnp.float32),
        mesh=mesh,
        compiler_params=pltpu.CompilerParams(needs_layout_passes=False),
        scratch_types=[
            pltpu.VMEM((2, SG, 128), jnp.int32),
            pltpu.VMEM((2, SG, 128), jnp.int32),
            pltpu.VMEM((RB,), jnp.int32),
            pltpu.VMEM((nrows, gs), jnp.int32),
            pltpu.VMEM((gs, w), jnp.float32),
            pltpu.VMEM((gs, w), jnp.float32),
            pltpu.VMEM_SHARED((ch + 8, w), jnp.float32),
            pltpu.SemaphoreType.DMA,
            pltpu.SemaphoreType.DMA,
            pltpu.SemaphoreType.DMA,
        ],
    )


def _scatter_sc(src, si, di, n_out):
    m = si.shape[0]
    w = src.shape[1]
    mult = 16 * SG * 128
    m_pad = -(-m // mult) * mult
    if m_pad != m:
        si = jnp.concatenate([si, jnp.zeros((m_pad - m,), jnp.int32)])
        di = jnp.concatenate([di, jnp.full((m_pad - m,), PAD_DI, jnp.int32)])
    ng = m_pad // 16 // 128
    ch = 11136 if n_out > 102400 else 10240
    n_ch = -(-n_out // ch)
    si3 = si.reshape(16, ng, 128)
    di3 = di.reshape(16, ng, 128)
    k = _make_scatter_kernel(ng, n_ch, w, ch)
    out = k(src, si3, di3)
    return out[:n_out]


def _linear_bn(x, w, b, scale=None, shift=None):
    n, din = x.shape
    dout = w.shape[1]
    nb = n // BN
    apply_act = scale is not None

    def body(*refs):
        if apply_act:
            x_ref, s_ref, h_ref, w_ref, b_ref, y_ref, st_ref = refs
            xx = jnp.maximum(x_ref[...] * s_ref[...] + h_ref[...], 0.0)
        else:
            x_ref, w_ref, b_ref, y_ref, st_ref = refs
            xx = x_ref[...]
        y = jnp.dot(xx, w_ref[...], preferred_element_type=jnp.float32) + b_ref[...]
        y_ref[...] = y
        st = jnp.stack([jnp.sum(y, axis=0), jnp.sum(y * y, axis=0)], axis=0)

        @pl.when(pl.program_id(0) == 0)
        def _():
            st_ref[...] = st

        @pl.when(pl.program_id(0) != 0)
        def _():
            st_ref[...] += st

    in_specs = [pl.BlockSpec((BN, din), lambda i: (i, 0))]
    args = [x]
    if apply_act:
        in_specs += [pl.BlockSpec((1, din), lambda i: (0, 0)),
                     pl.BlockSpec((1, din), lambda i: (0, 0))]
        args += [scale.reshape(1, din), shift.reshape(1, din)]
    in_specs += [pl.BlockSpec((din, dout), lambda i: (0, 0)),
                 pl.BlockSpec((1, dout), lambda i: (0, 0))]
    args += [w, b.reshape(1, dout)]

    y, st = pl.pallas_call(
        body,
        grid=(nb,),
        in_specs=in_specs,
        out_specs=[pl.BlockSpec((BN, dout), lambda i: (i, 0)),
                   pl.BlockSpec((2, dout), lambda i: (0, 0))],
        out_shape=[jax.ShapeDtypeStruct((n, dout), jnp.float32),
                   jax.ShapeDtypeStruct((2, dout), jnp.float32)],
    )(*args)
    return y, st


def _bn_coeffs(st, n, g, be):
    mu = st[0] / n
    var = st[1] / n - mu * mu
    inv = g * lax.rsqrt(var + BN_EPS)
    return inv, be - mu * inv


def _mlp(params, x):
    n = x.shape[0]
    scale = shift = None
    for (w, b, g, be) in params:
        x, st = _linear_bn(x, w, b, scale, shift)
        scale, shift = _bn_coeffs(st, n, g, be)
    return x, scale, shift


def _affine_relu(x, scale, shift):
    n, d = x.shape

    def body(x_ref, s_ref, h_ref, o_ref):
        o_ref[...] = jnp.maximum(x_ref[...] * s_ref[...] + h_ref[...], 0.0)

    return pl.pallas_call(
        body,
        grid=(n // BN,),
        in_specs=[pl.BlockSpec((BN, d), lambda i: (i, 0)),
                  pl.BlockSpec((1, d), lambda i: (0, 0)),
                  pl.BlockSpec((1, d), lambda i: (0, 0))],
        out_specs=pl.BlockSpec((BN, d), lambda i: (i, 0)),
        out_shape=jax.ShapeDtypeStruct((n, d), jnp.float32),
    )(x, scale.reshape(1, d), shift.reshape(1, d))


def _combine(x1, x2, x3, a, b):
    n, d = x1.shape

    def body(x1_ref, x2_ref, x3_ref, a_ref, b_ref, o_ref):
        o_ref[...] = (x1_ref[...] * a_ref[...] + x2_ref[...] * b_ref[...]
                      + x3_ref[...])

    sspec = pl.BlockSpec((1, 1), lambda i: (0, 0))
    bspec = pl.BlockSpec((BN, d), lambda i: (i, 0))
    return pl.pallas_call(
        body,
        grid=(n // BN,),
        in_specs=[bspec, bspec, bspec, sspec, sspec],
        out_specs=bspec,
        out_shape=jax.ShapeDtypeStruct((n, d), jnp.float32),
    )(x1, x2, x3, a.reshape(1, 1), b.reshape(1, 1))


def _autobahn(x, p_a, p_b):
    n, din = x.shape
    dout = p_a[0].shape[1]
    na = SPLIT // BN
    w2 = jnp.stack([p_a[0], p_b[0]])
    b2 = jnp.stack([p_a[1].reshape(1, dout), p_b[1].reshape(1, dout)])

    def body(x_ref, w_ref, b_ref, o_ref):
        o_ref[...] = (jnp.dot(x_ref[...], w_ref[0],
                              preferred_element_type=jnp.float32) + b_ref[0])

    def widx(i):
        s = jnp.where(i >= na, 1, 0)
        return (s, 0, 0)

    return pl.pallas_call(
        body,
        grid=(n // BN,),
        in_specs=[pl.BlockSpec((BN, din), lambda i: (i, 0)),
                  pl.BlockSpec((1, din, dout), widx),
                  pl.BlockSpec((1, 1, dout), widx)],
        out_specs=pl.BlockSpec((BN, dout), lambda i: (i, 0)),
        out_shape=jax.ShapeDtypeStruct((n, dout), jnp.float32),
    )(x, w2, b2)


def _segpool_part(y3):
    g, r, d = y3.shape
    gb = 1000

    def body(x_ref, o_ref):
        x = x_ref[...]
        o_ref[...] = jnp.broadcast_to(jnp.sum(x, axis=1, keepdims=True), x.shape)

    return pl.pallas_call(
        body,
        grid=(g // gb,),
        in_specs=[pl.BlockSpec((gb, r, d), lambda i: (i, 0, 0))],
        out_specs=pl.BlockSpec((gb, r, d), lambda i: (i, 0, 0)),
        out_shape=jax.ShapeDtypeStruct((g, r, d), jnp.float32),
    )(y3)


def _segpool_bcast(y):
    d = y.shape[1]
    p1 = _segpool_part(y[:SPLIT].reshape(10000, 5, d)).reshape(SPLIT, d)
    p2 = _segpool_part(y[SPLIT:].reshape(10000, 6, d)).reshape(C_ROWS - SPLIT, d)
    return jnp.concatenate([p1, p2], axis=0)


def _pipeline(edge_attr, cycle_attr, params, cycle_ids,
              e2c_src_1, e2c_dst_1, e2c_src_2, e2c_dst_2, c2c_src, c2c_dst,
              c2e_src_1, c2e_dst_1, c2e_src_2, c2e_dst_2, scatter):
    p = params
    one = jnp.float32(1.0)

    e2c1 = scatter(edge_attr, e2c_src_1, e2c_dst_1, C_ROWS)
    e2c2 = scatter(edge_attr, e2c_src_2, e2c_dst_2, C_ROWS)

    y2 = scatter(e2c2, c2c_src, c2c_dst, C_ROWS)
    y1 = scatter(e2c1, c2c_src, c2c_dst, C_ROWS)
    yc = scatter(cycle_attr, c2c_src, c2c_dst, C_ROWS)
    p2b = _segpool_bcast(y2)
    p1b = _segpool_bcast(y1)
    pcb = _segpool_bcast(yc)

    lift_in = jnp.concatenate([y2, p2b, y1, p1b], axis=-1)
    lift_raw, ls, lh = _mlp(p['cycle_mlp_2'], lift_in)
    lift_final = _affine_relu(lift_raw, ls, lh)

    ab1 = _autobahn(e2c1, p['ab_c5'], p['ab_c6'])
    ab2 = _autobahn(e2c2, p['ab_c5'], p['ab_c6'])
    ab_raw, as_, ah = _mlp(p['cycle_ab_mlp'], jnp.concatenate([ab1, ab2], axis=-1))
    e2c_ab = _affine_relu(ab_raw, as_, ah)

    cycle_lin = jnp.concatenate([yc, pcb], axis=-1)
    comb_c = _combine(cycle_lin, lift_final, e2c_ab,
                      one + p['eps_c1'], one + p['eps_c2'])
    cyc_raw, cs, chh = _mlp(p['cycle_mlp_1'], comb_c)
    cycle_out = _affine_relu(cyc_raw, cs, chh)

    lvl_raw, vs, vh = _mlp(p['edge_mlp_1'],
                           jnp.concatenate([lift_final, cycle_attr], axis=-1))
    lvl_final = _affine_relu(lvl_raw, vs, vh)

    ylvl = scatter(lvl_final, c2c_src, c2c_dst, C_ROWS)
    plvlb = _segpool_bcast(ylvl)
    lvl_ab = _autobahn(jnp.concatenate([ylvl, plvlb], axis=-1),
                       p['ab_l5'], p['ab_l6'])

    lv1a = scatter(ylvl, c2e_src_1, c2e_dst_1, E_ROWS)
    lv1b = scatter(plvlb, c2e_src_1, c2e_dst_1, E_ROWS)
    lv2a = scatter(ylvl, c2e_src_2, c2e_dst_2, E_ROWS)
    lv2b = scatter(plvlb, c2e_src_2, c2e_dst_2, E_ROWS)
    la1 = scatter(lvl_ab, c2e_src_1, c2e_dst_1, E_ROWS)
    la2 = scatter(lvl_ab, c2e_src_2, c2e_dst_2, E_ROWS)

    aggr_raw, gsc, gh = _mlp(p['edge_mlp_3'],
                             jnp.concatenate([lv1a, lv1b, lv2a, lv2b], axis=-1))
    aggr_final = _affine_relu(aggr_raw, gsc, gh)
    abt_raw, ts, th = _mlp(p['edge_mlp_4'],
                           jnp.concatenate([la1, la2], axis=-1))
    abt_final = _affine_relu(abt_raw, ts, th)

    comb_e = _combine(edge_attr, aggr_final, abt_final,
                      one + p['eps_e1'], one + p['eps_e2'])
    edge_raw, es, eh = _mlp(p['edge_mlp_2'], comb_e)
    edge_out = _affine_relu(edge_raw, es, eh)

    return (edge_out, cycle_out)


def kernel(edge_attr, cycle_attr, params, cycle_ids,
           e2c_src_1, e2c_dst_1, e2c_src_2, e2c_dst_2, c2c_src, c2c_dst,
           c2e_src_1, c2e_dst_1, c2e_src_2, c2e_dst_2):
    return _pipeline(edge_attr, cycle_attr, params, cycle_ids,
                     e2c_src_1, e2c_dst_1, e2c_src_2, e2c_dst_2,
                     c2c_src, c2c_dst,
                     c2e_src_1, c2e_dst_1, c2e_src_2, c2e_dst_2,
                     _scatter_sc)
